# baseline (speedup 1.0000x reference)
import jax
import jax.numpy as jnp
from jax import lax
from jax.experimental import pallas as pl
from jax.experimental.pallas import tpu as pltpu
from jax.experimental.pallas import tpu_sc as plsc

BATCH = 64
SEQ_LEN = 512
N_EMBD = 512

NC = 2
NS = 16
L = 16
NPG = 16
P_PER_W = SEQ_LEN // NPG
B_PER_W = BATCH // 2
NBUF = 4
LEAD = 3
CCHUNKS = N_EMBD // L


def _embed_body(x_hbm, tok_hbm, pos_hbm, out_hbm,
                idx_v, pos_v, gbuf, gsem, osem):
    wid = lax.axis_index("s") * NC + lax.axis_index("c")
    pg = wid // 2
    bh = wid % 2
    p0 = pg * P_PER_W
    b0 = bh * B_PER_W

    c0 = (pg // 4) * 128
    coff = (pg % 4) * P_PER_W
    pltpu.sync_copy(x_hbm.at[pl.ds(b0, B_PER_W), pl.ds(c0, 128)], idx_v)
    pltpu.sync_copy(pos_hbm.at[pl.ds(p0, P_PER_W), :], pos_v)

    def gather(t, slot):
        return pltpu.make_async_copy(
            tok_hbm.at[idx_v.at[t, pl.ds(coff, P_PER_W)]],
            gbuf.at[slot], gsem.at[slot])

    def out_dma(t, slot):
        return pltpu.make_async_copy(
            gbuf.at[slot], out_hbm.at[b0 + t, pl.ds(p0, P_PER_W), :],
            osem.at[slot])

    for k in range(LEAD):
        gather(k, k).start()

    def group(g, _):
        for k in range(NBUF):
            t = g * NBUF + k
            gather(t, k).wait()

            def add_chunk(c, _):
                cs = pl.ds(c * L, L)
                for p in range(P_PER_W):
                    gbuf[k, p, cs] = gbuf[k, p, cs] + pos_v[p, cs]
                return ()
            lax.fori_loop(0, CCHUNKS, add_chunk, ())

            out_dma(t, k).start()

            kg = (k + LEAD) % NBUF

            @pl.when(t + LEAD < B_PER_W)
            def _():
                @pl.when(t >= NBUF - LEAD)
                def _():
                    out_dma(t + LEAD - NBUF, kg).wait()
                gather(t + LEAD, kg).start()
        return ()

    lax.fori_loop(0, B_PER_W // NBUF, group, ())

    for t in range(B_PER_W - NBUF, B_PER_W):
        out_dma(t, t % NBUF).wait()


@jax.jit
def _embed(x, token_table, position_table):
    mesh = plsc.VectorSubcoreMesh(core_axis_name="c", subcore_axis_name="s")
    return pl.kernel(
        _embed_body,
        out_type=jax.ShapeDtypeStruct((BATCH, SEQ_LEN, N_EMBD), jnp.float32),
        mesh=mesh,
        scratch_types=[
            pltpu.VMEM((B_PER_W, 128), jnp.int32),
            pltpu.VMEM((P_PER_W, N_EMBD), jnp.float32),
            pltpu.VMEM((NBUF, P_PER_W, N_EMBD), jnp.float32),
            pltpu.SemaphoreType.DMA((NBUF,)),
            pltpu.SemaphoreType.DMA((NBUF,)),
        ],
    )(x, token_table, position_table)


def kernel(x, token_table, position_table):
    return _embed(x, token_table, position_table)


# speedup vs baseline: 1.1513x; 1.1513x over previous
import jax
import jax.numpy as jnp
from jax import lax
from jax.experimental import pallas as pl
from jax.experimental.pallas import tpu as pltpu
from jax.experimental.pallas import tpu_sc as plsc

BATCH = 64
SEQ_LEN = 512
N_EMBD = 512

NC = 2
NS = 16
L = 16
NW = NC * NS
P_PER_W = SEQ_LEN // NW
NBUF = 8
LEAD = 6
CCHUNKS = N_EMBD // L


def _embed_body(x_hbm, tok_hbm, pos_hbm, out_hbm,
                idx_v, pos_v, gbuf, gsem, osem):
    wid = lax.axis_index("s") * NC + lax.axis_index("c")
    p0 = wid * P_PER_W

    c0 = (wid // 8) * 128
    coff = (wid % 8) * P_PER_W
    pltpu.sync_copy(x_hbm.at[:, pl.ds(c0, 128)], idx_v)
    pltpu.sync_copy(pos_hbm.at[pl.ds(p0, P_PER_W), :], pos_v)

    def gather(b, slot):
        return pltpu.make_async_copy(
            tok_hbm.at[idx_v.at[b, pl.ds(coff, P_PER_W)]],
            gbuf.at[slot], gsem.at[slot])

    def out_dma(b, slot):
        return pltpu.make_async_copy(
            gbuf.at[slot], out_hbm.at[b, pl.ds(p0, P_PER_W), :],
            osem.at[slot])

    def out_dma_half(b, slot, h):
        hp = P_PER_W // 2
        return pltpu.make_async_copy(
            gbuf.at[slot, pl.ds(h * hp, hp)],
            out_hbm.at[b, pl.ds(p0 + h * hp, hp), :],
            osem.at[slot])

    for k in range(LEAD):
        gather(k, k).start()

    def group(g, _):
        for k in range(NBUF):
            b = g * NBUF + k
            gather(b, k).wait()

            for h in range(2):
                def add_chunk(c, _, h=h):
                    cs = pl.ds(c * L, L)
                    for p in range(h * P_PER_W // 2, (h + 1) * P_PER_W // 2):
                        gbuf[k, p, cs] = gbuf[k, p, cs] + pos_v[p, cs]
                    return ()
                lax.fori_loop(0, CCHUNKS, add_chunk, ())
                out_dma_half(b, k, h).start()

            kg = (k + LEAD) % NBUF

            @pl.when(b + LEAD < BATCH)
            def _():
                @pl.when(b >= NBUF - LEAD)
                def _():
                    out_dma(b + LEAD - NBUF, kg).wait()
                gather(b + LEAD, kg).start()
        return ()

    lax.fori_loop(0, BATCH // NBUF, group, ())

    for b in range(BATCH - NBUF, BATCH):
        out_dma(b, b % NBUF).wait()


@jax.jit
def _embed(x, token_table, position_table):
    mesh = plsc.VectorSubcoreMesh(core_axis_name="c", subcore_axis_name="s")
    return pl.kernel(
        _embed_body,
        out_type=jax.ShapeDtypeStruct((BATCH, SEQ_LEN, N_EMBD), jnp.float32),
        mesh=mesh,
        scratch_types=[
            pltpu.VMEM((BATCH, 128), jnp.int32),
            pltpu.VMEM((P_PER_W, N_EMBD), jnp.float32),
            pltpu.VMEM((NBUF, P_PER_W, N_EMBD), jnp.float32),
            pltpu.SemaphoreType.DMA((NBUF,)),
            pltpu.SemaphoreType.DMA((NBUF,)),
        ],
    )(x, token_table, position_table)


def kernel(x, token_table, position_table):
    return _embed(x, token_table, position_table)


# speedup vs baseline: 1.1789x; 1.0240x over previous
import jax
import jax.numpy as jnp
from jax import lax
from jax.experimental import pallas as pl
from jax.experimental.pallas import tpu as pltpu
from jax.experimental.pallas import tpu_sc as plsc

BATCH = 64
SEQ_LEN = 512
N_EMBD = 512

NC = 2
NS = 16
L = 16
NW = NC * NS
P_PER_W = SEQ_LEN // NW
NBUF = 12
LEAD = 8
NMAIN = (BATCH // NBUF) * NBUF
CCHUNKS = N_EMBD // L


def _embed_body(x_hbm, tok_hbm, pos_hbm, out_hbm,
                idx_v, pos_v, gbuf, gsem, osem):
    wid = lax.axis_index("s") * NC + lax.axis_index("c")
    p0 = wid * P_PER_W

    c0 = (wid // 8) * 128
    coff = (wid % 8) * P_PER_W
    pltpu.sync_copy(x_hbm.at[:, pl.ds(c0, 128)], idx_v)
    pltpu.sync_copy(pos_hbm.at[pl.ds(p0, P_PER_W), :], pos_v)

    def gather(b, slot):
        return pltpu.make_async_copy(
            tok_hbm.at[idx_v.at[b, pl.ds(coff, P_PER_W)]],
            gbuf.at[slot], gsem.at[slot])

    def out_dma(b, slot):
        return pltpu.make_async_copy(
            gbuf.at[slot], out_hbm.at[b, pl.ds(p0, P_PER_W), :],
            osem.at[slot])

    for k in range(LEAD):
        gather(k, k).start()

    def group(g, _):
        for k in range(NBUF):
            b = g * NBUF + k
            gather(b, k).wait()

            def add_chunk(c, _):
                cs = pl.ds(c * L, L)
                for p in range(P_PER_W):
                    gbuf[k, p, cs] = gbuf[k, p, cs] + pos_v[p, cs]
                return ()
            lax.fori_loop(0, CCHUNKS, add_chunk, ())

            out_dma(b, k).start()

            kg = (k + LEAD) % NBUF

            @pl.when(b + LEAD < BATCH)
            def _():
                @pl.when(b >= NBUF - LEAD)
                def _():
                    out_dma(b + LEAD - NBUF, kg).wait()
                gather(b + LEAD, kg).start()
        return ()

    lax.fori_loop(0, BATCH // NBUF, group, ())

    for b in range(NMAIN, BATCH):
        k = b % NBUF
        gather(b, k).wait()

        def add_chunk(c, _, k=k):
            cs = pl.ds(c * L, L)
            for p in range(P_PER_W):
                gbuf[k, p, cs] = gbuf[k, p, cs] + pos_v[p, cs]
            return ()
        lax.fori_loop(0, CCHUNKS, add_chunk, ())
        out_dma(b, k).start()

    for b in range(BATCH - NBUF, BATCH):
        out_dma(b, b % NBUF).wait()


@jax.jit
def _embed(x, token_table, position_table):
    mesh = plsc.VectorSubcoreMesh(core_axis_name="c", subcore_axis_name="s")
    return pl.kernel(
        _embed_body,
        out_type=jax.ShapeDtypeStruct((BATCH, SEQ_LEN, N_EMBD), jnp.float32),
        mesh=mesh,
        scratch_types=[
            pltpu.VMEM((BATCH, 128), jnp.int32),
            pltpu.VMEM((P_PER_W, N_EMBD), jnp.float32),
            pltpu.VMEM((NBUF, P_PER_W, N_EMBD), jnp.float32),
            pltpu.SemaphoreType.DMA((NBUF,)),
            pltpu.SemaphoreType.DMA((NBUF,)),
        ],
    )(x, token_table, position_table)


def kernel(x, token_table, position_table):
    return _embed(x, token_table, position_table)


# speedup vs baseline: 1.1790x; 1.0001x over previous
import jax
import jax.numpy as jnp
from jax import lax
from jax.experimental import pallas as pl
from jax.experimental.pallas import tpu as pltpu
from jax.experimental.pallas import tpu_sc as plsc

BATCH = 64
SEQ_LEN = 512
N_EMBD = 512

NC = 2
NS = 16
L = 16
NW = NC * NS
P_PER_W = SEQ_LEN // NW
NBUF = 12
LEAD = 8
NMAIN = (BATCH // NBUF) * NBUF
CCHUNKS = N_EMBD // L


def _embed_body(x_hbm, tok_hbm, pos_hbm, out_hbm,
                idx_v, pos_v, gbuf, gsem, osem):
    wid = lax.axis_index("s") * NC + lax.axis_index("c")
    p0 = wid * P_PER_W

    c0 = (wid // 8) * 128
    coff = (wid % 8) * P_PER_W
    pltpu.sync_copy(x_hbm.at[:, pl.ds(c0, 128)], idx_v)
    pltpu.sync_copy(pos_hbm.at[pl.ds(p0, P_PER_W), :], pos_v)

    def gather(b, slot):
        return pltpu.make_async_copy(
            tok_hbm.at[idx_v.at[b, pl.ds(coff, P_PER_W)]],
            gbuf.at[slot], gsem.at[slot])

    def out_dma(b, slot):
        return pltpu.make_async_copy(
            gbuf.at[slot], out_hbm.at[b, pl.ds(p0, P_PER_W), :],
            osem.at[slot])

    for k in range(LEAD):
        gather(k, k).start()

    def group(g, _):
        for k in range(NBUF):
            b = g * NBUF + k
            gather(b, k).wait()

            kg = (k + LEAD) % NBUF

            @pl.when(b + LEAD < BATCH)
            def _():
                @pl.when(b >= NBUF - LEAD)
                def _():
                    out_dma(b + LEAD - NBUF, kg).wait()
                gather(b + LEAD, kg).start()

            def add_chunk(c, _):
                cs = pl.ds(c * L, L)
                for p in range(P_PER_W):
                    gbuf[k, p, cs] = gbuf[k, p, cs] + pos_v[p, cs]
                return ()
            lax.fori_loop(0, CCHUNKS, add_chunk, ())

            out_dma(b, k).start()
        return ()

    lax.fori_loop(0, BATCH // NBUF, group, ())

    for b in range(NMAIN, BATCH):
        k = b % NBUF
        gather(b, k).wait()

        def add_chunk(c, _, k=k):
            cs = pl.ds(c * L, L)
            for p in range(P_PER_W):
                gbuf[k, p, cs] = gbuf[k, p, cs] + pos_v[p, cs]
            return ()
        lax.fori_loop(0, CCHUNKS, add_chunk, ())
        out_dma(b, k).start()

    for b in range(BATCH - NBUF, BATCH):
        out_dma(b, b % NBUF).wait()


@jax.jit
def _embed(x, token_table, position_table):
    mesh = plsc.VectorSubcoreMesh(core_axis_name="c", subcore_axis_name="s")
    return pl.kernel(
        _embed_body,
        out_type=jax.ShapeDtypeStruct((BATCH, SEQ_LEN, N_EMBD), jnp.float32),
        mesh=mesh,
        scratch_types=[
            pltpu.VMEM((BATCH, 128), jnp.int32),
            pltpu.VMEM((P_PER_W, N_EMBD), jnp.float32),
            pltpu.VMEM((NBUF, P_PER_W, N_EMBD), jnp.float32),
            pltpu.SemaphoreType.DMA((NBUF,)),
            pltpu.SemaphoreType.DMA((NBUF,)),
        ],
    )(x, token_table, position_table)


def kernel(x, token_table, position_table):
    return _embed(x, token_table, position_table)
